# Initial kernel scaffold; baseline (speedup 1.0000x reference)
import jax
import jax.numpy as jnp
from jax.experimental import pallas as pl


def kernel(input_ids, cr0, cr1, cr2, ci0, ci1, ci2):
    raise NotImplementedError("write your pallas kernel here")


# speedup vs baseline: 19.8305x; 19.8305x over previous
import functools

import jax
import jax.numpy as jnp
from jax import lax
from jax.experimental import pallas as pl
from jax.experimental.pallas import tpu as pltpu
from jax.experimental.pallas import tpu_sc as plsc

V1, V2, V3 = 20, 40, 40
D1, D2, D3 = 4, 4, 8
RR = 4
NROWS = V1 * V2 * V3
DREAL = D1 * D2 * D3
DOUT = 2 * DREAL
ROWS_PER_I = V2 * V3
GH = RR * D3
RFH = RR * D2 * D3


def _prep_branch(c0, c1, c2):
    a = c0[0]
    b = c1
    c = c2[..., 0]
    eye_f = jnp.eye(D2, dtype=jnp.float32)
    eye_h = jnp.eye(D3, dtype=jnp.float32)
    cmat = c.transpose(1, 0, 2).reshape(V3, GH)
    v = jnp.einsum('rjfg,hq->jghrfq', b, eye_h).reshape(V2, GH, RFH)
    w = jnp.einsum('idr,fp,hq->irfhdpq', a, eye_f, eye_h).reshape(V1, RFH, DREAL)
    return cmat, v, w


def _table_body(cmr, vr, cmi, vi, wr, wi, out_ref, qr, qi):
    i = pl.program_id(0)

    @pl.when(i == 0)
    def _():
        for j in range(V2):
            qr[pl.ds(j * V3, V3), :] = jnp.dot(
                cmr[...], vr[j], preferred_element_type=jnp.float32)
            qi[pl.ds(j * V3, V3), :] = jnp.dot(
                cmi[...], vi[j], preferred_element_type=jnp.float32)

    out_ref[:, :DREAL] = jnp.dot(qr[...], wr[0],
                                 preferred_element_type=jnp.float32)
    out_ref[:, DREAL:] = jnp.dot(qi[...], wi[0],
                                 preferred_element_type=jnp.float32)


def _build_table(cmr, vr, wr, cmi, vi, wi):
    return pl.pallas_call(
        _table_body,
        grid=(V1,),
        in_specs=[
            pl.BlockSpec((V3, GH), lambda i: (0, 0)),
            pl.BlockSpec((V2, GH, RFH), lambda i: (0, 0, 0)),
            pl.BlockSpec((V3, GH), lambda i: (0, 0)),
            pl.BlockSpec((V2, GH, RFH), lambda i: (0, 0, 0)),
            pl.BlockSpec((1, RFH, DREAL), lambda i: (i, 0, 0)),
            pl.BlockSpec((1, RFH, DREAL), lambda i: (i, 0, 0)),
        ],
        out_specs=pl.BlockSpec((ROWS_PER_I, DOUT), lambda i: (i, 0)),
        out_shape=jax.ShapeDtypeStruct((NROWS, DOUT), jnp.float32),
        scratch_shapes=[
            pltpu.VMEM((ROWS_PER_I, RFH), jnp.float32),
            pltpu.VMEM((ROWS_PER_I, RFH), jnp.float32),
        ],
    )(cmr, vr, cmi, vi, wr, wi)


CHUNK = 128


def _make_gather(n_tokens):
    info = plsc.get_sparse_core_info()
    nw = info.num_cores * info.num_subcores
    per_w = n_tokens // nw
    n_chunks = per_w // CHUNK
    assert per_w * nw == n_tokens and n_chunks * CHUNK == per_w
    mesh = plsc.VectorSubcoreMesh(core_axis_name="c", subcore_axis_name="s")

    @functools.partial(
        pl.kernel,
        mesh=mesh,
        out_type=jax.ShapeDtypeStruct((n_tokens, DOUT), jnp.float32),
        scratch_types=[
            pltpu.VMEM((per_w,), jnp.int32),
            pltpu.VMEM((CHUNK, DOUT), jnp.float32),
            pltpu.SemaphoreType.DMA,
        ],
    )
    def gather_k(ids_hbm, table_hbm, out_hbm, idx_v, rows_v, sem):
        wid = lax.axis_index("s") * info.num_cores + lax.axis_index("c")
        base = pl.multiple_of(wid * per_w, per_w)
        pltpu.sync_copy(ids_hbm.at[pl.ds(base, per_w)], idx_v)

        def mod_body(t, carry):
            off = pl.multiple_of(t * 16, 16)
            idx_v[pl.ds(off, 16)] = lax.rem(idx_v[pl.ds(off, 16)], NROWS)
            return carry

        lax.fori_loop(0, per_w // 16, mod_body, 0)

        def chunk_body(g, carry):
            off = pl.multiple_of(g * CHUNK, CHUNK)
            pltpu.async_copy(
                table_hbm.at[idx_v.at[pl.ds(off, CHUNK)]], rows_v, sem
            ).wait()
            pltpu.sync_copy(rows_v, out_hbm.at[pl.ds(base + off, CHUNK)])
            return carry

        lax.fori_loop(0, n_chunks, chunk_body, 0)

    return gather_k


def kernel(input_ids, cr0, cr1, cr2, ci0, ci1, ci2):
    B, S = input_ids.shape
    ids = input_ids.reshape(-1).astype(jnp.int32)
    cmr, vr, wr = _prep_branch(cr0, cr1, cr2)
    cmi, vi, wi = _prep_branch(ci0, ci1, ci2)
    table = _build_table(cmr, vr, wr, cmi, vi, wi)
    out = _make_gather(B * S)(ids, table)
    return out.reshape(B, S, DOUT)


# speedup vs baseline: 20.8287x; 1.0503x over previous
import functools

import jax
import jax.numpy as jnp
from jax import lax
from jax.experimental import pallas as pl
from jax.experimental.pallas import tpu as pltpu
from jax.experimental.pallas import tpu_sc as plsc

V1, V2, V3 = 20, 40, 40
D1, D2, D3 = 4, 4, 8
RR = 4
NROWS = V1 * V2 * V3
DREAL = D1 * D2 * D3
DOUT = 2 * DREAL
ROWS_PER_I = V2 * V3
GH = RR * D3
RFH = RR * D2 * D3


def _prep_branch(c0, c1, c2):
    a = c0[0]
    b = c1
    c = c2[..., 0]
    eye_f = jnp.eye(D2, dtype=jnp.float32)
    eye_h = jnp.eye(D3, dtype=jnp.float32)
    cmat = c.transpose(1, 0, 2).reshape(V3, GH)
    v = jnp.einsum('rjfg,hq->jghrfq', b, eye_h).reshape(V2, GH, RFH)
    w = jnp.einsum('idr,fp,hq->irfhdpq', a, eye_f, eye_h).reshape(V1, RFH, DREAL)
    return cmat, v, w


def _table_body(cmr, vr, cmi, vi, wr, wi, out_ref, qr, qi):
    i = pl.program_id(0)

    @pl.when(i == 0)
    def _():
        for j in range(V2):
            qr[pl.ds(j * V3, V3), :] = jnp.dot(
                cmr[...], vr[j], preferred_element_type=jnp.float32)
            qi[pl.ds(j * V3, V3), :] = jnp.dot(
                cmi[...], vi[j], preferred_element_type=jnp.float32)

    out_ref[:, :DREAL] = jnp.dot(qr[...], wr[0],
                                 preferred_element_type=jnp.float32)
    out_ref[:, DREAL:] = jnp.dot(qi[...], wi[0],
                                 preferred_element_type=jnp.float32)


def _build_table(cmr, vr, wr, cmi, vi, wi):
    return pl.pallas_call(
        _table_body,
        grid=(V1,),
        in_specs=[
            pl.BlockSpec((V3, GH), lambda i: (0, 0)),
            pl.BlockSpec((V2, GH, RFH), lambda i: (0, 0, 0)),
            pl.BlockSpec((V3, GH), lambda i: (0, 0)),
            pl.BlockSpec((V2, GH, RFH), lambda i: (0, 0, 0)),
            pl.BlockSpec((1, RFH, DREAL), lambda i: (i, 0, 0)),
            pl.BlockSpec((1, RFH, DREAL), lambda i: (i, 0, 0)),
        ],
        out_specs=pl.BlockSpec((ROWS_PER_I, DOUT), lambda i: (i, 0)),
        out_shape=jax.ShapeDtypeStruct((NROWS, DOUT), jnp.float32),
        scratch_shapes=[
            pltpu.VMEM((ROWS_PER_I, RFH), jnp.float32),
            pltpu.VMEM((ROWS_PER_I, RFH), jnp.float32),
        ],
    )(cmr, vr, cmi, vi, wr, wi)


CHUNK = 128


def _make_gather(n_tokens):
    info = plsc.get_sparse_core_info()
    nw = info.num_cores * info.num_subcores
    per_w = n_tokens // nw
    n_chunks = per_w // CHUNK
    assert per_w * nw == n_tokens and n_chunks * CHUNK == per_w
    mesh = plsc.VectorSubcoreMesh(core_axis_name="c", subcore_axis_name="s")

    assert n_chunks % 2 == 0

    @functools.partial(
        pl.kernel,
        mesh=mesh,
        out_type=jax.ShapeDtypeStruct((n_tokens, DOUT), jnp.float32),
        scratch_types=[
            pltpu.VMEM((per_w,), jnp.int32),
            pltpu.VMEM((2, CHUNK, DOUT), jnp.float32),
            pltpu.SemaphoreType.DMA,
            pltpu.SemaphoreType.DMA,
            pltpu.SemaphoreType.DMA,
            pltpu.SemaphoreType.DMA,
        ],
    )
    def gather_k(ids_hbm, table_hbm, out_hbm, idx_v, rows_v, sg0, sg1, ss0, ss1):
        sem_g = (sg0, sg1)
        sem_s = (ss0, ss1)
        wid = lax.axis_index("s") * info.num_cores + lax.axis_index("c")
        base = pl.multiple_of(wid * per_w, per_w)
        pltpu.sync_copy(ids_hbm.at[pl.ds(base, per_w)], idx_v)

        def mod_slice(t, carry):
            off = pl.multiple_of(t * 16, 16)
            idx_v[pl.ds(off, 16)] = lax.rem(idx_v[pl.ds(off, 16)], NROWS)
            return carry

        def g_copy(g, b):
            off = pl.multiple_of(g * CHUNK, CHUNK)
            return pltpu.make_async_copy(
                table_hbm.at[idx_v.at[pl.ds(off, CHUNK)]], rows_v.at[b],
                sem_g[b])

        def s_copy(g, b):
            off = pl.multiple_of(g * CHUNK, CHUNK)
            return pltpu.make_async_copy(
                rows_v.at[b], out_hbm.at[pl.ds(base + off, CHUNK)], sem_s[b])

        lax.fori_loop(0, CHUNK // 16, mod_slice, 0)
        g_copy(0, 0).start()
        lax.fori_loop(CHUNK // 16, per_w // 16, mod_slice, 0)

        def body(t, carry):
            for b in range(2):
                g = t * 2 + b
                g_copy(g, b).wait()
                s_copy(g, b).start()

                @pl.when(g >= 1)
                def _():
                    s_copy(g - 1, 1 - b).wait()

                @pl.when(g + 1 < n_chunks)
                def _():
                    g_copy(g + 1, 1 - b).start()

            return carry

        lax.fori_loop(0, n_chunks // 2, body, 0)
        s_copy(n_chunks - 1, (n_chunks - 1) % 2).wait()

    return gather_k


def kernel(input_ids, cr0, cr1, cr2, ci0, ci1, ci2):
    B, S = input_ids.shape
    ids = input_ids.reshape(-1).astype(jnp.int32)
    cmr, vr, wr = _prep_branch(cr0, cr1, cr2)
    cmi, vi, wi = _prep_branch(ci0, ci1, ci2)
    table = _build_table(cmr, vr, wr, cmi, vi, wi)
    out = _make_gather(B * S)(ids, table)
    return out.reshape(B, S, DOUT)
